# Initial kernel scaffold; baseline (speedup 1.0000x reference)
import jax
import jax.numpy as jnp
from jax.experimental import pallas as pl


def kernel(tokens, seq_ids, num_tokens, is_boundary, max_boundaries):
    raise NotImplementedError("write your pallas kernel here")


# speedup vs baseline: 1.3404x; 1.3404x over previous
import functools

import jax
import jax.numpy as jnp
from jax import lax
from jax.experimental import pallas as pl
from jax.experimental.pallas import tpu as pltpu
from jax.experimental.pallas import tpu_sc as plsc

P = 32768
L = 16
NSUB = 16
CHUNK = P // NSUB
GROUP = 128
NGROUP = CHUNK // GROUP
MAXB = 16


def _sc_body(mask_hbm, mb_hbm, out_hbm, xchg_hbm, chunk_v, pos_v, all_v,
             out_v, mb_v):
    wid = lax.axis_index("s")
    base = wid * CHUNK
    pltpu.sync_copy(mask_hbm.at[pl.ds(base, CHUNK)], chunk_v)
    pos_v[...] = jnp.full((L,), -1, jnp.int32)
    iota = lax.iota(jnp.int32, L)

    def group_body(g, cnt):
        o = g * GROUP
        s = chunk_v[pl.ds(o, L)]
        for j in range(1, GROUP // L):
            s = s + chunk_v[pl.ds(o + j * L, L)]
        c = jnp.sum(s)

        def slow(cc):
            for j in range(GROUP // L):
                v = chunk_v[pl.ds(o + j * L, L)]
                incl = plsc.cumsum(v)
                tgt = cc + incl - 1
                posv = base + o + j * L + iota
                plsc.store_scatter(pos_v, [tgt], posv,
                                   mask=(v != 0) & (tgt < L))
                cc = cc + jnp.sum(v)
            return cc

        return lax.cond(c > 0, slow, lambda cc: cc, cnt)

    lax.fori_loop(0, NGROUP, group_body, jnp.int32(0))

    pltpu.sync_copy(pos_v, xchg_hbm.at[wid])
    plsc.subcore_barrier()

    @pl.when(wid == 0)
    def _():
        pltpu.sync_copy(xchg_hbm, all_v)
        pltpu.sync_copy(mb_hbm, mb_v)
        out_v[...] = jnp.full((L,), -1, jnp.int32)
        mb = mb_v[...]
        off = jnp.int32(0)
        for t in range(NSUB):
            vec = all_v[t]
            m = vec >= 0
            tgt = off + iota
            plsc.store_scatter(out_v, [tgt], vec,
                               mask=m & (tgt < L) & (tgt < mb))
            off = off + jnp.sum(m.astype(jnp.int32))
        pltpu.sync_copy(out_v, out_hbm)


@functools.lru_cache(maxsize=1)
def _sc_compact():
    return pl.kernel(
        _sc_body,
        out_type=(jax.ShapeDtypeStruct((MAXB,), jnp.int32),
                  jax.ShapeDtypeStruct((NSUB, L), jnp.int32)),
        mesh=plsc.VectorSubcoreMesh(
            core_axis_name="c", subcore_axis_name="s",
            num_cores=1, num_subcores=NSUB),
        scratch_types=[
            pltpu.VMEM((CHUNK,), jnp.int32),
            pltpu.VMEM((L,), jnp.int32),
            pltpu.VMEM((NSUB, L), jnp.int32),
            pltpu.VMEM((L,), jnp.int32),
            pltpu.VMEM((L,), jnp.int32),
        ],
        compiler_params=pltpu.CompilerParams(needs_layout_passes=False),
    )


def kernel(tokens, seq_ids, num_tokens, is_boundary, max_boundaries):
    mask_i32 = is_boundary.astype(jnp.int32)
    mb_vec = jnp.full((MAXB,), max_boundaries, dtype=jnp.int32)
    return _sc_compact()(mask_i32, mb_vec)[0]
